# baseline (speedup 1.0000x reference)
import functools

import jax
import jax.numpy as jnp
from jax import lax
from jax.experimental import pallas as pl
from jax.experimental.pallas import tpu as pltpu
from jax.experimental.pallas import tpu_sc as plsc

NUM_UNITS = 64
SCALE = 8.0
NW = 32
CHUNK = 128
LANES = 16


def _sc_body(idx_hbm, table_hbm, out_hbm, idx_v, rows_v, gsem):
    nchunk = idx_v.shape[0] // CHUNK
    wid = lax.axis_index("s") * 2 + lax.axis_index("c")
    pltpu.sync_copy(idx_hbm.at[wid], idx_v)

    def chunk_body(j, carry):
        base = j * CHUNK
        pltpu.async_copy(
            table_hbm.at[idx_v.at[pl.ds(base, CHUNK)]], rows_v, gsem
        ).wait()
        for rg in range(CHUNK // LANES):
            iv = idx_v[pl.ds(base + rg * LANES, LANES)]
            m16 = jnp.where(iv == 0, 0.0, SCALE).astype(jnp.float32)
            for i in range(LANES):
                r = rg * LANES + i
                m = jnp.full((LANES,), m16[i], dtype=jnp.float32)
                for c in range(0, NUM_UNITS, LANES):
                    rows_v[r, pl.ds(c, LANES)] = rows_v[r, pl.ds(c, LANES)] * m
        pltpu.sync_copy(rows_v, out_hbm.at[wid, j])
        return carry

    lax.fori_loop(0, nchunk, chunk_body, 0)


def kernel(inputs, shared_weights):
    n_tok = inputs.shape[0] * inputs.shape[1]
    per_w = n_tok // NW
    nchunk = per_w // CHUNK
    idx3 = inputs.reshape(NW, per_w)

    mesh = plsc.VectorSubcoreMesh(core_axis_name="c", subcore_axis_name="s")
    run = pl.kernel(
        _sc_body,
        out_type=jax.ShapeDtypeStruct((NW, nchunk, CHUNK, NUM_UNITS), jnp.float32),
        mesh=mesh,
        scratch_types=[
            pltpu.VMEM((per_w,), jnp.int32),
            pltpu.VMEM((CHUNK, NUM_UNITS), jnp.float32),
            pltpu.SemaphoreType.DMA,
        ],
        compiler_params=pltpu.CompilerParams(use_tc_tiling_on_sc=False),
    )
    out = run(idx3, shared_weights)
    return out.reshape(inputs.shape[0], inputs.shape[1], NUM_UNITS)


# speedup vs baseline: 1.1049x; 1.1049x over previous
import functools

import jax
import jax.numpy as jnp
from jax import lax
from jax.experimental import pallas as pl
from jax.experimental.pallas import tpu as pltpu
from jax.experimental.pallas import tpu_sc as plsc

NUM_UNITS = 64
SCALE = 8.0
NW = 32
CHUNK = 128
LANES = 16


NBUF = 8
PF = 4


def _sc_body(idx_hbm, table_hbm, out_hbm, idx_v, rows_v, gsem, ssem):
    nchunk = idx_v.shape[0] // CHUNK
    wid = lax.axis_index("s") * 2 + lax.axis_index("c")
    pltpu.sync_copy(idx_hbm.at[wid], idx_v)

    def start_gather(j, b):
        pltpu.async_copy(
            table_hbm.at[idx_v.at[pl.ds(j * CHUNK, CHUNK)]],
            rows_v.at[b],
            gsem.at[b],
        )

    for b in range(PF):
        start_gather(b, b)

    def outer(j0, carry):
        for b in range(NBUF):
            j = j0 + b
            bn = (b + PF) % NBUF
            jn = j + PF

            @pl.when(jn < nchunk)
            def _():
                @pl.when(jn >= NBUF)
                def _():
                    pltpu.make_async_copy(
                        rows_v.at[bn], out_hbm.at[wid, 0], ssem.at[bn]
                    ).wait()

                start_gather(jn, bn)

            pltpu.make_async_copy(
                table_hbm.at[idx_v.at[pl.ds(j * CHUNK, CHUNK)]],
                rows_v.at[b],
                gsem.at[b],
            ).wait()
            for rg in range(CHUNK // LANES):
                iv = idx_v[pl.ds(j * CHUNK + rg * LANES, LANES)]
                m16 = jnp.where(iv == 0, 0.0, SCALE).astype(jnp.float32)
                for i in range(LANES):
                    r = rg * LANES + i
                    m = jnp.full((LANES,), m16[i], dtype=jnp.float32)
                    for c in range(0, NUM_UNITS, LANES):
                        rows_v[b, r, pl.ds(c, LANES)] = (
                            rows_v[b, r, pl.ds(c, LANES)] * m
                        )
            pltpu.async_copy(rows_v.at[b], out_hbm.at[wid, j], ssem.at[b])
        return carry

    lax.fori_loop(0, nchunk // NBUF, lambda t, c: outer(t * NBUF, c), 0)

    for b in range(NBUF):
        pltpu.make_async_copy(
            rows_v.at[b], out_hbm.at[wid, 0], ssem.at[b]
        ).wait()


def kernel(inputs, shared_weights):
    n_tok = inputs.shape[0] * inputs.shape[1]
    per_w = n_tok // NW
    nchunk = per_w // CHUNK
    idx3 = inputs.reshape(NW, per_w)

    mesh = plsc.VectorSubcoreMesh(core_axis_name="c", subcore_axis_name="s")
    run = pl.kernel(
        _sc_body,
        out_type=jax.ShapeDtypeStruct((NW, nchunk, CHUNK, NUM_UNITS), jnp.float32),
        mesh=mesh,
        scratch_types=[
            pltpu.VMEM((per_w,), jnp.int32),
            pltpu.VMEM((NBUF, CHUNK, NUM_UNITS), jnp.float32),
            pltpu.SemaphoreType.DMA((NBUF,)),
            pltpu.SemaphoreType.DMA((NBUF,)),
        ],
        compiler_params=pltpu.CompilerParams(use_tc_tiling_on_sc=False),
    )
    out = run(idx3, shared_weights)
    return out.reshape(inputs.shape[0], inputs.shape[1], NUM_UNITS)
